# baseline (speedup 1.0000x reference)
import functools

import jax
import jax.numpy as jnp
import numpy as np
from jax import lax
from jax.experimental import pallas as pl
from jax.experimental.pallas import tpu as pltpu
from jax.experimental.pallas import tpu_sc as plsc

_NC = 2
_NS = 16
_K = 80


def _ea_body(ea_ref, we_ref, be_ref, out_ref):
    r2, d2 = out_ref.shape
    res = (
        jnp.dot(ea_ref[...], we_ref[...], preferred_element_type=jnp.float32)
        + be_ref[...]
    )
    out_ref[...] = res.astype(jnp.bfloat16).reshape(r2, d2)


def _edge_proj(edge_attr, we, be):
    e, de = edge_attr.shape
    d = we.shape[1]
    be_block = 4000
    return pl.pallas_call(
        _ea_body,
        grid=(e // be_block,),
        in_specs=[
            pl.BlockSpec((be_block, de), lambda i: (i, 0)),
            pl.BlockSpec((de, d), lambda i: (0, 0)),
            pl.BlockSpec((1, d), lambda i: (0, 0)),
        ],
        out_specs=pl.BlockSpec((be_block // 2, 2 * d), lambda i: (i, 0)),
        out_shape=jax.ShapeDtypeStruct((e // 2, 2 * d), jnp.bfloat16),
    )(edge_attr, we, be.reshape(1, d))


def _mlp_body(h_ref, a0_ref, a1_ref, w1_ref, b1_ref, w2_ref, b2_ref, g_ref,
              bb_ref, out_ref):
    h = h_ref[...]
    z = h + a0_ref[...] + a1_ref[...]
    t = jnp.maximum(
        jnp.dot(z, w1_ref[...], preferred_element_type=jnp.float32) + b1_ref[...],
        0.0,
    )
    o = jnp.dot(t, w2_ref[...], preferred_element_type=jnp.float32) + b2_ref[...]
    m = jnp.mean(o, axis=-1, keepdims=True)
    v = jnp.mean((o - m) ** 2, axis=-1, keepdims=True)
    o = (o - m) / jnp.sqrt(v + 1e-5) * g_ref[...] + bb_ref[...]
    out_ref[...] = jnp.maximum(o + h, 0.0)


def _mlp(h, a0, a1, w1, b1, w2, b2, g, bb):
    n, d = h.shape
    dh = w1.shape[1]
    bn = 2000
    full = lambda i: (0, 0)
    row = lambda i: (i, 0)
    return pl.pallas_call(
        _mlp_body,
        grid=(n // bn,),
        in_specs=[
            pl.BlockSpec((bn, d), row),
            pl.BlockSpec((bn, d), row),
            pl.BlockSpec((bn, d), row),
            pl.BlockSpec((d, dh), full),
            pl.BlockSpec((1, dh), full),
            pl.BlockSpec((dh, d), full),
            pl.BlockSpec((1, d), full),
            pl.BlockSpec((1, d), full),
            pl.BlockSpec((1, d), full),
        ],
        out_specs=pl.BlockSpec((bn, d), row),
        out_shape=jax.ShapeDtypeStruct((n, d), jnp.float32),
    )(h, a0, a1, w1, b1.reshape(1, dh), w2, b2.reshape(1, d), g.reshape(1, d),
      bb.reshape(1, d))


def _make_sc_agg(n, e, d):
    ept = e // (_NC * _NS)
    chunks = ept // _K
    npad = -(-n // (8 * _NS)) * (8 * _NS)
    zr = npad // _NS
    mesh = plsc.VectorSubcoreMesh(core_axis_name="c", subcore_axis_name="s")

    vec = []
    vec += [pltpu.VMEM((_K,), jnp.int32) for _ in range(4)]
    vec += [pltpu.VMEM((_K,), jnp.int32) for _ in range(2)]
    vec += [pltpu.VMEM((_K, d), jnp.float32) for _ in range(2)]
    vec += [pltpu.VMEM((_K // 2, d), jnp.int32) for _ in range(2)]
    vec += [pltpu.SemaphoreType.DMA for _ in range(12)]
    vec += [pltpu.VMEM_SHARED((npad, d), jnp.float32)]

    @functools.partial(
        pl.kernel,
        mesh=mesh,
        out_type=jax.ShapeDtypeStruct((_NC, npad, d), jnp.float32),
        scratch_types=vec,
    )
    def sc_agg(h_hbm, src_hbm, dst_hbm, ea_hbm, out_hbm, *bufs):
        sidx = [bufs[0:2], bufs[2:4]]
        didx = bufs[4:6]
        rows = bufs[6:8]
        eab = bufs[8:10]
        semg = bufs[10:12]
        seme = bufs[12:14]
        sems = bufs[14:16]
        semi = [bufs[16:18], bufs[18:20]]
        semid = bufs[20:22]
        agg_sp = bufs[22]
        c = lax.axis_index("c")
        s = lax.axis_index("s")
        wid = c * _NS + s
        ebase = wid * ept

        def zrow(k, carry):
            for j in range(d // 16):
                rows[0][k, pl.ds(j * 16, 16)] = jnp.zeros((16,), jnp.float32)
            return carry

        lax.fori_loop(0, _K, zrow, 0)
        zbase = s * zr
        for t in range(zr // _K):
            pltpu.sync_copy(rows[0], agg_sp.at[pl.ds(zbase + t * _K, _K)])
        if zr % _K != 0:
            pltpu.sync_copy(rows[0], agg_sp.at[pl.ds(zbase + zr - _K, _K)])
        plsc.subcore_barrier()

        def issue(i, b, p):
            g = pltpu.async_copy(h_hbm.at[sidx[b][p]], rows[b], semg[b])
            ec = pltpu.async_copy(
                ea_hbm.at[pl.ds(pl.multiple_of((ebase + i * _K) // 2, 8),
                                _K // 2)],
                eab[b], seme[b])
            return g, ec

        def compute(b):
            mask = jnp.full((16,), -65536, jnp.int32)

            def mrow(k2, carry2):
                for half in range(2):
                    k = 2 * k2 + half
                    for g in range(d // 32):
                        ve = eab[b][k2, pl.ds((d // 2) * half + 16 * g, 16)]
                        lo = lax.bitcast_convert_type(ve << 16, jnp.float32)
                        hi = lax.bitcast_convert_type(ve & mask, jnp.float32)
                        sl = pl.ds(32 * g, 16)
                        sh = pl.ds(32 * g + 16, 16)
                        rows[b][k, sl] = jnp.maximum(rows[b][k, sl] + lo, 0.0)
                        rows[b][k, sh] = jnp.maximum(rows[b][k, sh] + hi, 0.0)
                return carry2

            lax.fori_loop(0, _K // 2, mrow, 0)

        def sidx_load(i, b, p):
            return pltpu.async_copy(src_hbm.at[pl.ds(ebase + i * _K, _K)],
                                    sidx[b][p], semi[b][p])

        def sidx_drain(b, p):
            pltpu.make_async_copy(src_hbm.at[pl.ds(ebase, _K)], sidx[b][p],
                                  semi[b][p]).wait()

        def didx_load(i, b):
            return pltpu.async_copy(dst_hbm.at[pl.ds(ebase + i * _K, _K)],
                                    didx[b], semid[b])

        def didx_drain(b):
            pltpu.make_async_copy(dst_hbm.at[pl.ds(ebase, _K)], didx[b],
                                  semid[b]).wait()

        pltpu.sync_copy(src_hbm.at[pl.ds(ebase, _K)], sidx[0][0])
        pltpu.sync_copy(src_hbm.at[pl.ds(ebase + _K, _K)], sidx[1][0])
        sidx_load(2, 0, 1)
        sidx_load(3, 1, 1)
        didx_load(0, 0)
        didx_load(1, 1)
        g0, e0 = issue(0, 0, 0)
        g1, e1 = issue(1, 1, 0)

        def quarter(u, j):
            c = 4 * u + j
            b = j % 2
            p = j // 2
            g0.wait() if b == 0 else g1.wait()
            e0.wait() if b == 0 else e1.wait()

            def _prefetch():
                sidx_load(c + 4, b, p)

            def _refill():
                didx_load(c + 2, b)
                sidx_drain(b, p ^ 1)
                issue(c + 2, b, p ^ 1)

            static = isinstance(c, int)
            if static:
                if c + 4 < chunks:
                    _prefetch()
            else:
                pl.when(c + 4 < chunks)(_prefetch)
            compute(b)
            didx_drain(b)
            sc = pltpu.async_copy(rows[b], agg_sp.at[didx[b]], sems[b],
                                  add=True)
            sc.wait()
            if static:
                if c + 2 < chunks:
                    _refill()
            else:
                pl.when(c + 2 < chunks)(_refill)

        def body(u, carry):
            for j in range(4):
                quarter(u, j)
            return carry

        lax.fori_loop(0, chunks // 4, body, 0)
        for j in range(chunks % 4):
            quarter(chunks // 4, j)
        plsc.subcore_barrier()
        pltpu.sync_copy(agg_sp.at[pl.ds(s * zr, zr)],
                        out_hbm.at[c, pl.ds(s * zr, zr)])

    return sc_agg


def kernel(x, edge_index, edge_attr, We, be, W1, b1, W2, b2, ln_g, ln_b):
    n, d = x.shape
    e = edge_index.shape[1]
    nl = We.shape[0]
    src = edge_index[0]
    dst = edge_index[1]
    sc_agg = _make_sc_agg(n, e, d)

    m = np.arange(d)
    q = 32 * (m // 32) + 16 * (m % 2) + (m % 32) // 2
    a = jnp.asarray(np.eye(d, dtype=np.float32)[q].T)

    def as_i32(t):
        return lax.bitcast_convert_type(
            t.reshape(t.shape[0], t.shape[1] // 2, 2), jnp.int32)

    h = x.astype(jnp.float32)
    eas = [as_i32(_edge_proj(edge_attr, We[i] @ a, be[i] @ a))
           for i in range(nl)]
    for i in range(nl):
        parts = sc_agg(h, src, dst, eas[i])
        h = _mlp(h, parts[0, :n], parts[1, :n], W1[i], b1[i], W2[i],
                 b2[i], ln_g[i], ln_b[i])
    return h


# speedup vs baseline: 3.4113x; 3.4113x over previous
import functools

import jax
import jax.numpy as jnp
from jax import lax
from jax.experimental import pallas as pl
from jax.experimental.pallas import tpu as pltpu
from jax.experimental.pallas import tpu_sc as plsc

_NC = 2
_NS = 16
_K = 80


def _ea_body(ea_ref, we_ref, be_ref, out_ref):
    out_ref[...] = (
        jnp.dot(ea_ref[...], we_ref[...], preferred_element_type=jnp.float32)
        + be_ref[...]
    )


def _edge_proj(edge_attr, we, be):
    e, de = edge_attr.shape
    d = we.shape[1]
    be_block = 4000
    return pl.pallas_call(
        _ea_body,
        grid=(e // be_block,),
        in_specs=[
            pl.BlockSpec((be_block, de), lambda i: (i, 0)),
            pl.BlockSpec((de, d), lambda i: (0, 0)),
            pl.BlockSpec((1, d), lambda i: (0, 0)),
        ],
        out_specs=pl.BlockSpec((be_block, d), lambda i: (i, 0)),
        out_shape=jax.ShapeDtypeStruct((e, d), jnp.float32),
    )(edge_attr, we, be.reshape(1, d))


def _mlp_body(h_ref, a0_ref, a1_ref, w1_ref, b1_ref, w2_ref, b2_ref, g_ref,
              bb_ref, out_ref):
    h = h_ref[...]
    z = h + a0_ref[...] + a1_ref[...]
    t = jnp.maximum(
        jnp.dot(z, w1_ref[...], preferred_element_type=jnp.float32) + b1_ref[...],
        0.0,
    )
    o = jnp.dot(t, w2_ref[...], preferred_element_type=jnp.float32) + b2_ref[...]
    m = jnp.mean(o, axis=-1, keepdims=True)
    v = jnp.mean((o - m) ** 2, axis=-1, keepdims=True)
    o = (o - m) / jnp.sqrt(v + 1e-5) * g_ref[...] + bb_ref[...]
    out_ref[...] = jnp.maximum(o + h, 0.0)


def _mlp(h, a0, a1, w1, b1, w2, b2, g, bb):
    n, d = h.shape
    dh = w1.shape[1]
    bn = 1000
    full = lambda i: (0, 0)
    row = lambda i: (i, 0)
    return pl.pallas_call(
        _mlp_body,
        grid=(n // bn,),
        in_specs=[
            pl.BlockSpec((bn, d), row),
            pl.BlockSpec((bn, d), row),
            pl.BlockSpec((bn, d), row),
            pl.BlockSpec((d, dh), full),
            pl.BlockSpec((1, dh), full),
            pl.BlockSpec((dh, d), full),
            pl.BlockSpec((1, d), full),
            pl.BlockSpec((1, d), full),
            pl.BlockSpec((1, d), full),
        ],
        out_specs=pl.BlockSpec((bn, d), row),
        out_shape=jax.ShapeDtypeStruct((n, d), jnp.float32),
    )(h, a0, a1, w1, b1.reshape(1, dh), w2, b2.reshape(1, d), g.reshape(1, d),
      bb.reshape(1, d))


def _make_sc_agg(n, e, d):
    ept = e // (_NC * _NS)
    chunks = ept // _K
    npad = -(-n // (8 * _NS)) * (8 * _NS)
    zr = npad // _NS
    mesh = plsc.VectorSubcoreMesh(core_axis_name="c", subcore_axis_name="s")

    vec = []
    vec += [pltpu.VMEM((_K,), jnp.int32) for _ in range(4)]
    vec += [pltpu.VMEM((_K,), jnp.int32) for _ in range(2)]
    vec += [pltpu.VMEM((_K, d), jnp.float32) for _ in range(2)]
    vec += [pltpu.VMEM((_K, d), jnp.float32) for _ in range(2)]
    vec += [pltpu.SemaphoreType.DMA for _ in range(12)]
    vec += [pltpu.VMEM_SHARED((npad, d), jnp.float32)]

    @functools.partial(
        pl.kernel,
        mesh=mesh,
        out_type=jax.ShapeDtypeStruct((_NC, npad, d), jnp.float32),
        scratch_types=vec,
    )
    def sc_agg(h_hbm, src_hbm, dst_hbm, ea_hbm, out_hbm, *bufs):
        sidx = [bufs[0:2], bufs[2:4]]
        didx = bufs[4:6]
        rows = bufs[6:8]
        msg = bufs[8:10]
        semg = bufs[10:12]
        seme = bufs[12:14]
        sems = bufs[14:16]
        semi = [bufs[16:18], bufs[18:20]]
        semid = bufs[20:22]
        agg_sp = bufs[22]
        c = lax.axis_index("c")
        s = lax.axis_index("s")
        wid = c * _NS + s
        ebase = wid * ept

        def zrow(k, carry):
            for j in range(d // 16):
                rows[0][k, pl.ds(j * 16, 16)] = jnp.zeros((16,), jnp.float32)
            return carry

        lax.fori_loop(0, _K, zrow, 0)
        zbase = s * zr
        for t in range(zr // _K):
            pltpu.sync_copy(rows[0], agg_sp.at[pl.ds(zbase + t * _K, _K)])
        if zr % _K != 0:
            pltpu.sync_copy(rows[0], agg_sp.at[pl.ds(zbase + zr - _K, _K)])
        plsc.subcore_barrier()

        def issue(i, b, p):
            g = pltpu.async_copy(h_hbm.at[sidx[b][p]], rows[b], semg[b])
            e = pltpu.async_copy(ea_hbm.at[pl.ds(ebase + i * _K, _K)],
                                 msg[b], seme[b])
            return g, e

        def compute(b):
            def mrow(k, carry2):
                for j in range(d // 16):
                    sl = pl.ds(j * 16, 16)
                    msg[b][k, sl] = jnp.maximum(msg[b][k, sl] + rows[b][k, sl],
                                                0.0)
                return carry2

            lax.fori_loop(0, _K, mrow, 0)

        def sidx_load(i, b, p):
            return pltpu.async_copy(src_hbm.at[pl.ds(ebase + i * _K, _K)],
                                    sidx[b][p], semi[b][p])

        def sidx_drain(b, p):
            pltpu.make_async_copy(src_hbm.at[pl.ds(ebase, _K)], sidx[b][p],
                                  semi[b][p]).wait()

        def didx_load(i, b):
            return pltpu.async_copy(dst_hbm.at[pl.ds(ebase + i * _K, _K)],
                                    didx[b], semid[b])

        def didx_drain(b):
            pltpu.make_async_copy(dst_hbm.at[pl.ds(ebase, _K)], didx[b],
                                  semid[b]).wait()

        pltpu.sync_copy(src_hbm.at[pl.ds(ebase, _K)], sidx[0][0])
        pltpu.sync_copy(src_hbm.at[pl.ds(ebase + _K, _K)], sidx[1][0])
        sidx_load(2, 0, 1)
        sidx_load(3, 1, 1)
        didx_load(0, 0)
        didx_load(1, 1)
        g0, e0 = issue(0, 0, 0)
        g1, e1 = issue(1, 1, 0)

        def quarter(u, j):
            c = 4 * u + j
            b = j % 2
            p = j // 2
            g0.wait() if b == 0 else g1.wait()
            e0.wait() if b == 0 else e1.wait()

            def _prefetch():
                sidx_load(c + 4, b, p)

            def _refill():
                didx_load(c + 2, b)
                sidx_drain(b, p ^ 1)
                issue(c + 2, b, p ^ 1)

            static = isinstance(c, int)
            if static:
                if c + 4 < chunks:
                    _prefetch()
            else:
                pl.when(c + 4 < chunks)(_prefetch)
            compute(b)
            didx_drain(b)
            sc = pltpu.async_copy(msg[b], agg_sp.at[didx[b]], sems[b],
                                  add=True)
            sc.wait()
            if static:
                if c + 2 < chunks:
                    _refill()
            else:
                pl.when(c + 2 < chunks)(_refill)

        def body(u, carry):
            for j in range(4):
                quarter(u, j)
            return carry

        lax.fori_loop(0, chunks // 4, body, 0)
        for j in range(chunks % 4):
            quarter(chunks // 4, j)
        plsc.subcore_barrier()
        pltpu.sync_copy(agg_sp.at[pl.ds(s * zr, zr)],
                        out_hbm.at[c, pl.ds(s * zr, zr)])

    return sc_agg


def kernel(x, edge_index, edge_attr, We, be, W1, b1, W2, b2, ln_g, ln_b):
    n, d = x.shape
    e = edge_index.shape[1]
    nl = We.shape[0]
    src = edge_index[0]
    dst = edge_index[1]
    sc_agg = _make_sc_agg(n, e, d)
    h = x.astype(jnp.float32)
    eas = [_edge_proj(edge_attr, We[i], be[i]) for i in range(nl)]
    for i in range(nl):
        parts = sc_agg(h, src, dst, eas[i])
        h = _mlp(h, parts[0, :n], parts[1, :n], W1[i], b1[i], W2[i], b2[i],
                 ln_g[i], ln_b[i])
    return h
